# baseline (speedup 1.0000x reference)
import functools

import jax
import jax.numpy as jnp
import numpy as np
from jax import lax
from jax.experimental import pallas as pl
from jax.experimental.pallas import tpu as pltpu
from jax.experimental.pallas import tpu_sc as plsc

IN_CH = 96
K_IM = 48
H = 224
W = 224
R = 32
NH = H // R
KS = 7
PAD = 3
EPS = 1e-5


def _sc_topk_mask(m_hbm, sel_hbm, m_v, out_v):
    nc = plsc.get_sparse_core_info().num_cores
    wid = lax.axis_index("s") * nc + lax.axis_index("c")
    nb = sel_hbm.shape[0]

    @pl.when(wid < nb)
    def _worker():
        pltpu.sync_copy(m_hbm.at[wid], m_v)
        nv = IN_CH // 16
        mcs = [m_v[pl.ds(cv * 16, 16)] for cv in range(nv)]
        cidx = [m_v[pl.ds(IN_CH * 17 + cv * 16, 16)] for cv in range(nv)]
        ranks = [jnp.zeros((16,), jnp.float32) for _ in range(nv)]
        for j in range(IN_CH):
            mj = m_v[pl.ds(IN_CH + j * 16, 16)]
            for cv in range(nv):
                ranks[cv] = (ranks[cv]
                             + jnp.where(mj > mcs[cv], 1.0, 0.0)
                             + jnp.where((mj == mcs[cv]) & (cidx[cv] > j),
                                         1.0, 0.0))
        for cv in range(nv):
            out_v[pl.ds(cv * 16, 16)] = jnp.where(ranks[cv] < K_IM, 1.0, 0.0)
        pltpu.sync_copy(out_v, sel_hbm.at[wid])


def _fused_kernel(x_ref, sel_ref, w_ref, b_ref, g_ref, bt_ref,
                  oim_ref, osub_ref, xprev_ref, maps_ref, cs_ref):
    b = pl.program_id(0)
    n = pl.program_id(1)

    @pl.when(n == 0)
    def _per_batch_setup():
        zslot = jnp.zeros((4, R, W), jnp.float32)
        maps_ref[:, 0] = zslot
        maps_ref[:, NH + 1] = zslot

    @pl.when(n < NH)
    def _reduce():
        s_im = mx_im = s_sub = mx_sub = None
        for c in range(IN_CH):
            si = sel_ref[b, c]
            xi = x_ref[0, c]
            mi = xi * si
            mo = xi - mi
            if c == 0:
                s_im, mx_im, s_sub, mx_sub = mi, mi, mo, mo
            else:
                s_im = s_im + mi
                mx_im = jnp.maximum(mx_im, mi)
                s_sub = s_sub + mo
                mx_sub = jnp.maximum(mx_sub, mo)
        maps_ref[0, n + 1] = s_im * (1.0 / K_IM)
        maps_ref[1, n + 1] = s_sub * (1.0 / (IN_CH - K_IM))
        maps_ref[2, n + 1] = mx_im
        maps_ref[3, n + 1] = mx_sub

    @pl.when(n > 0)
    def _apply():
        top = maps_ref[:, n - 1, R - PAD:R, :]
        mid = maps_ref[:, n]
        bot = maps_ref[:, n + 1, 0:PAD, :]
        wu = jnp.concatenate([top, mid, bot], axis=1)
        for kw in range(KS):
            d = kw - PAD
            if d < 0:
                sh = jnp.concatenate(
                    [jnp.zeros((4, R + 2 * PAD, -d), jnp.float32),
                     wu[:, :, :W + d]], axis=2)
            elif d > 0:
                sh = jnp.concatenate(
                    [wu[:, :, d:],
                     jnp.zeros((4, R + 2 * PAD, d), jnp.float32)], axis=2)
            else:
                sh = wu
            cs_ref[kw] = sh

        scale = g_ref[0] * np.float32(1.0 / np.sqrt(1.0 + EPS))
        bias = b_ref[0]
        beta = bt_ref[0]
        acc = jnp.zeros((2, R, W), jnp.float32)
        for ci in range(2):
            for kh in range(KS):
                for kw in range(KS):
                    acc += w_ref[0, ci, kh, kw] * cs_ref[kw, 2 * ci:2 * ci + 2,
                                                         kh:kh + R, :]
        h = (acc + bias) * scale + beta
        att = jax.nn.sigmoid(jax.nn.relu(h))
        att_im = att[0]
        att_sub = att[1]

        for c in range(IN_CH):
            si = sel_ref[b, c]
            xi = xprev_ref[c]
            mi = xi * si
            oim_ref[0, c] = mi * att_im
            osub_ref[0, c] = (xi - mi) * att_sub

    @pl.when(n < NH)
    def _retain_x():
        xprev_ref[...] = x_ref[0]


@jax.jit
def kernel(x, M, conv_w, conv_b, bn_gamma, bn_beta):
    B = x.shape[0]
    f32 = jnp.float32

    mflat = M.reshape(B, IN_CH)
    mrep = jnp.broadcast_to(mflat[:, :, None], (B, IN_CH, 16))
    ramp = jnp.broadcast_to(jnp.arange(IN_CH, dtype=f32), (B, IN_CH))
    marg = jnp.concatenate([mflat, mrep.reshape(B, IN_CH * 16), ramp], axis=1)
    sel = functools.partial(
        pl.kernel,
        mesh=plsc.VectorSubcoreMesh(core_axis_name="c", subcore_axis_name="s"),
        out_type=jax.ShapeDtypeStruct((B, IN_CH), f32),
        scratch_types=[
            pltpu.VMEM((IN_CH * 18,), f32),
            pltpu.VMEM((IN_CH,), f32),
        ],
    )(_sc_topk_mask)(marg)

    out_im, out_sub = pl.pallas_call(
        _fused_kernel,
        grid=(B, NH + 1),
        compiler_params=pltpu.CompilerParams(
            dimension_semantics=("parallel", "arbitrary")),
        in_specs=[
            pl.BlockSpec((1, IN_CH, R, W),
                         lambda b, n: (b, 0, jnp.minimum(n, NH - 1), 0)),
            pl.BlockSpec(memory_space=pltpu.SMEM),
            pl.BlockSpec((1, 2, KS, KS), lambda b, n: (0, 0, 0, 0)),
            pl.BlockSpec(memory_space=pltpu.SMEM),
            pl.BlockSpec(memory_space=pltpu.SMEM),
            pl.BlockSpec(memory_space=pltpu.SMEM),
        ],
        out_specs=[
            pl.BlockSpec((1, IN_CH, R, W),
                         lambda b, n: (b, 0, jnp.maximum(n - 1, 0), 0)),
            pl.BlockSpec((1, IN_CH, R, W),
                         lambda b, n: (b, 0, jnp.maximum(n - 1, 0), 0)),
        ],
        out_shape=[
            jax.ShapeDtypeStruct((B, IN_CH, H, W), f32),
            jax.ShapeDtypeStruct((B, IN_CH, H, W), f32),
        ],
        scratch_shapes=[
            pltpu.VMEM((IN_CH, R, W), f32),
            pltpu.VMEM((4, NH + 2, R, W), f32),
            pltpu.VMEM((KS, 4, R + 2 * PAD, W), f32),
        ],
    )(x, sel, conv_w, conv_b, bn_gamma, bn_beta)

    return (out_im, out_sub)


# speedup vs baseline: 1.0218x; 1.0218x over previous
import functools

import jax
import jax.numpy as jnp
import numpy as np
from jax import lax
from jax.experimental import pallas as pl
from jax.experimental.pallas import tpu as pltpu
from jax.experimental.pallas import tpu_sc as plsc

IN_CH = 96
K_IM = 48
H = 224
W = 224
R = 32
NH = H // R
KS = 7
PAD = 3
EPS = 1e-5


def _sc_topk_mask(m_hbm, sel_hbm, m_v, out_v):
    nc = plsc.get_sparse_core_info().num_cores
    wid = lax.axis_index("s") * nc + lax.axis_index("c")
    nb = sel_hbm.shape[0]
    nv = IN_CH // 16
    bb = wid // nv
    cv = wid % nv

    @pl.when(wid < nb * nv)
    def _worker():
        pltpu.sync_copy(m_hbm.at[bb], m_v)
        mc = m_v[pl.ds(cv * 16, 16)]
        cidx = m_v[pl.ds(IN_CH * 17 + cv * 16, 16)]
        rank = jnp.zeros((16,), jnp.float32)
        for j in range(IN_CH):
            mj = m_v[pl.ds(IN_CH + j * 16, 16)]
            rank = (rank
                    + jnp.where(mj > mc, 1.0, 0.0)
                    + jnp.where((mj == mc) & (cidx > j), 1.0, 0.0))
        out_v[...] = jnp.where(rank < K_IM, 1.0, 0.0)
        pltpu.sync_copy(out_v, sel_hbm.at[bb, pl.ds(cv * 16, 16)])


def _fused_kernel(x_ref, sel_ref, w_ref, b_ref, g_ref, bt_ref,
                  oim_ref, osub_ref, xprev_ref, maps_ref, cs_ref):
    b = pl.program_id(0)
    n = pl.program_id(1)

    @pl.when(n == 0)
    def _per_batch_setup():
        zslot = jnp.zeros((4, R, W), jnp.float32)
        maps_ref[:, 0] = zslot
        maps_ref[:, NH + 1] = zslot

    @pl.when(n < NH)
    def _reduce():
        s_im = mx_im = s_sub = mx_sub = None
        for c in range(IN_CH):
            si = sel_ref[b, c]
            xi = x_ref[0, c]
            mi = xi * si
            mo = xi - mi
            if c == 0:
                s_im, mx_im, s_sub, mx_sub = mi, mi, mo, mo
            else:
                s_im = s_im + mi
                mx_im = jnp.maximum(mx_im, mi)
                s_sub = s_sub + mo
                mx_sub = jnp.maximum(mx_sub, mo)
        maps_ref[0, n + 1] = s_im * (1.0 / K_IM)
        maps_ref[1, n + 1] = s_sub * (1.0 / (IN_CH - K_IM))
        maps_ref[2, n + 1] = mx_im
        maps_ref[3, n + 1] = mx_sub

    @pl.when(n > 0)
    def _apply():
        top = maps_ref[:, n - 1, R - PAD:R, :]
        mid = maps_ref[:, n]
        bot = maps_ref[:, n + 1, 0:PAD, :]
        wu = jnp.concatenate([top, mid, bot], axis=1)
        for kw in range(KS):
            d = kw - PAD
            if d < 0:
                sh = jnp.concatenate(
                    [jnp.zeros((4, R + 2 * PAD, -d), jnp.float32),
                     wu[:, :, :W + d]], axis=2)
            elif d > 0:
                sh = jnp.concatenate(
                    [wu[:, :, d:],
                     jnp.zeros((4, R + 2 * PAD, d), jnp.float32)], axis=2)
            else:
                sh = wu
            cs_ref[kw] = sh

        scale = g_ref[0] * np.float32(1.0 / np.sqrt(1.0 + EPS))
        bias = b_ref[0]
        beta = bt_ref[0]
        acc = jnp.zeros((2, R, W), jnp.float32)
        for ci in range(2):
            for kh in range(KS):
                for kw in range(KS):
                    acc += w_ref[0, ci, kh, kw] * cs_ref[kw, 2 * ci:2 * ci + 2,
                                                         kh:kh + R, :]
        h = (acc + bias) * scale + beta
        att = jax.nn.sigmoid(jax.nn.relu(h))
        att_im = att[0]
        att_sub = att[1]

        for c in range(IN_CH):
            si = sel_ref[b, c]
            xi = xprev_ref[c]
            mi = xi * si
            oim_ref[0, c] = mi * att_im
            osub_ref[0, c] = (xi - mi) * att_sub

    @pl.when(n < NH)
    def _retain_x():
        xprev_ref[...] = x_ref[0]


@jax.jit
def kernel(x, M, conv_w, conv_b, bn_gamma, bn_beta):
    B = x.shape[0]
    f32 = jnp.float32

    mflat = M.reshape(B, IN_CH)
    mrep = jnp.broadcast_to(mflat[:, :, None], (B, IN_CH, 16))
    ramp = jnp.broadcast_to(jnp.arange(IN_CH, dtype=f32), (B, IN_CH))
    marg = jnp.concatenate([mflat, mrep.reshape(B, IN_CH * 16), ramp], axis=1)
    sel = functools.partial(
        pl.kernel,
        mesh=plsc.VectorSubcoreMesh(core_axis_name="c", subcore_axis_name="s"),
        out_type=jax.ShapeDtypeStruct((B, IN_CH), f32),
        scratch_types=[
            pltpu.VMEM((IN_CH * 18,), f32),
            pltpu.VMEM((16,), f32),
        ],
    )(_sc_topk_mask)(marg)

    out_im, out_sub = pl.pallas_call(
        _fused_kernel,
        grid=(B, NH + 1),
        compiler_params=pltpu.CompilerParams(
            dimension_semantics=("parallel", "arbitrary")),
        in_specs=[
            pl.BlockSpec((1, IN_CH, R, W),
                         lambda b, n: (b, 0, jnp.minimum(n, NH - 1), 0)),
            pl.BlockSpec(memory_space=pltpu.SMEM),
            pl.BlockSpec((1, 2, KS, KS), lambda b, n: (0, 0, 0, 0)),
            pl.BlockSpec(memory_space=pltpu.SMEM),
            pl.BlockSpec(memory_space=pltpu.SMEM),
            pl.BlockSpec(memory_space=pltpu.SMEM),
        ],
        out_specs=[
            pl.BlockSpec((1, IN_CH, R, W),
                         lambda b, n: (b, 0, jnp.maximum(n - 1, 0), 0)),
            pl.BlockSpec((1, IN_CH, R, W),
                         lambda b, n: (b, 0, jnp.maximum(n - 1, 0), 0)),
        ],
        out_shape=[
            jax.ShapeDtypeStruct((B, IN_CH, H, W), f32),
            jax.ShapeDtypeStruct((B, IN_CH, H, W), f32),
        ],
        scratch_shapes=[
            pltpu.VMEM((IN_CH, R, W), f32),
            pltpu.VMEM((4, NH + 2, R, W), f32),
            pltpu.VMEM((KS, 4, R + 2 * PAD, W), f32),
        ],
    )(x, sel, conv_w, conv_b, bn_gamma, bn_beta)

    return (out_im, out_sub)
